# baseline (speedup 1.0000x reference)
import functools

import jax
import jax.numpy as jnp
from jax import lax
from jax.experimental import pallas as pl
from jax.experimental.pallas import tpu as pltpu
from jax.experimental.pallas import tpu_sc as plsc

NUM_CORES = 2
NUM_SUBCORES = 16
NUM_WORKERS = NUM_CORES * NUM_SUBCORES
CHUNK = 128


@functools.partial(jax.jit, static_argnums=(2, 3))
def _gather_flat(idx2d, table, chunks_total, dim):
    chunks_per_w = chunks_total // NUM_WORKERS

    mesh = plsc.VectorSubcoreMesh(core_axis_name="c", subcore_axis_name="s")

    @functools.partial(
        pl.kernel,
        mesh=mesh,
        compiler_params=pltpu.CompilerParams(use_tc_tiling_on_sc=False),
        out_type=jax.ShapeDtypeStruct((chunks_total * CHUNK, dim), jnp.float32),
        scratch_types=[
            pltpu.VMEM((chunks_per_w, CHUNK), jnp.int32),
            pltpu.VMEM((CHUNK, dim), jnp.float32),
            pltpu.SemaphoreType.DMA,
        ],
    )
    def k(idx_hbm, table_hbm, out_hbm, idx_v, rows_v, gsem):
        c = lax.axis_index("c")
        s = lax.axis_index("s")
        wid = s * NUM_CORES + c
        cbase = wid * chunks_per_w
        pltpu.sync_copy(idx_hbm.at[pl.ds(cbase, chunks_per_w)], idx_v)

        def body(j, carry):
            pltpu.async_copy(table_hbm.at[idx_v.at[j]], rows_v, gsem).wait()
            pltpu.sync_copy(
                rows_v, out_hbm.at[pl.ds((cbase + j) * CHUNK, CHUNK)]
            )
            return carry

        lax.fori_loop(0, chunks_per_w, body, 0)

    return k(idx2d, table)


def kernel(token_ids, embedding_matrix):
    b, s = token_ids.shape
    v, d = embedding_matrix.shape
    total = b * s
    assert total % (NUM_WORKERS * CHUNK) == 0
    idx2d = token_ids.reshape(total // CHUNK, CHUNK).astype(jnp.int32)
    out = _gather_flat(idx2d, embedding_matrix, total // CHUNK, d)
    return out.reshape(b, s, d)


# speedup vs baseline: 1.1164x; 1.1164x over previous
import functools

import jax
import jax.numpy as jnp
from jax import lax
from jax.experimental import pallas as pl
from jax.experimental.pallas import tpu as pltpu
from jax.experimental.pallas import tpu_sc as plsc

NUM_CORES = 2
NUM_SUBCORES = 16
NUM_WORKERS = NUM_CORES * NUM_SUBCORES
CHUNK = 128
NBUF = 8


@functools.partial(jax.jit, static_argnums=(2, 3))
def _gather_flat(idx2d, table, chunks_total, dim):
    chunks_per_w = chunks_total // NUM_WORKERS
    ngroups = chunks_per_w // NBUF

    mesh = plsc.VectorSubcoreMesh(core_axis_name="c", subcore_axis_name="s")

    @functools.partial(
        pl.kernel,
        mesh=mesh,
        compiler_params=pltpu.CompilerParams(use_tc_tiling_on_sc=False),
        out_type=jax.ShapeDtypeStruct((chunks_total * CHUNK, dim), jnp.float32),
        scratch_types=[
            pltpu.VMEM((chunks_per_w, CHUNK), jnp.int32),
            pltpu.VMEM((NBUF, CHUNK, dim), jnp.float32),
            pltpu.SemaphoreType.DMA((NBUF,)),
            pltpu.SemaphoreType.DMA((NBUF,)),
        ],
    )
    def k(idx_hbm, table_hbm, out_hbm, idx_v, rows_v, gsem, osem):
        c = lax.axis_index("c")
        s = lax.axis_index("s")
        wid = s * NUM_CORES + c
        cbase = wid * chunks_per_w
        pltpu.sync_copy(idx_hbm.at[pl.ds(cbase, chunks_per_w)], idx_v)

        def gather(chunk, b):
            pltpu.async_copy(
                table_hbm.at[idx_v.at[chunk]], rows_v.at[b], gsem.at[b]
            )

        def writeback(chunk, b):
            pltpu.async_copy(
                rows_v.at[b],
                out_hbm.at[pl.ds((cbase + chunk) * CHUNK, CHUNK)],
                osem.at[b],
            )

        def wait_gather(b):
            pltpu.make_async_copy(
                table_hbm.at[pl.ds(0, CHUNK)], rows_v.at[b], gsem.at[b]
            ).wait()

        def wait_out(b):
            pltpu.make_async_copy(
                rows_v.at[b], out_hbm.at[pl.ds(0, CHUNK)], osem.at[b]
            ).wait()

        for b in range(NBUF):
            gather(b, b)

        def body(g, carry):
            for b in range(NBUF):
                wait_gather(b)
                writeback(g * NBUF + b, b)
            for b in range(NBUF):
                wait_out(b)
                gather((g + 1) * NBUF + b, b)
            return carry

        lax.fori_loop(0, ngroups - 1, body, 0)

        for b in range(NBUF):
            wait_gather(b)
            writeback((ngroups - 1) * NBUF + b, b)
        for b in range(NBUF):
            wait_out(b)

    return k(idx2d, table)


def kernel(token_ids, embedding_matrix):
    b, s = token_ids.shape
    v, d = embedding_matrix.shape
    total = b * s
    assert total % (NUM_WORKERS * CHUNK * NBUF) == 0
    idx2d = token_ids.reshape(total // CHUNK, CHUNK).astype(jnp.int32)
    out = _gather_flat(idx2d, embedding_matrix, total // CHUNK, d)
    return out.reshape(b, s, d)
